# baseline (speedup 1.0000x reference)
import functools

import jax
import jax.numpy as jnp
from jax import lax
from jax.experimental import pallas as pl
from jax.experimental.pallas import tpu as pltpu
from jax.experimental.pallas import tpu_sc as plsc

E = 8
TOP_K = 2
D = 1024
DH = D // 2
PC = DH // 128
D_FF = 512
T = 2048
BT = 256
P = 6144
NB = P // BT
NA = T * TOP_K


def _silu(x):
    return x * jax.nn.sigmoid(x)


def _cumsum_manual(y, axis):
    n = y.shape[axis]
    s = 1
    while s < n:
        if axis == 1:
            sh = jnp.concatenate([jnp.zeros((y.shape[0], s), y.dtype), y[:, :-s]], axis=1)
        else:
            sh = jnp.concatenate([jnp.zeros((s, y.shape[1]), y.dtype), y[:-s, :]], axis=0)
        y = y + sh
        s *= 2
    return y


def _to_slabs(a):
    w = a.shape[1]
    return jnp.concatenate(
        [a[:, None, c * 128:(c + 1) * 128] for c in range(w // 128)], axis=1)


def _from_slabs(a3):
    return jnp.concatenate([a3[:, c, :] for c in range(a3.shape[1])], axis=1)


def _pack2(a):
    lo = lax.bitcast_convert_type(a[:, :DH].astype(jnp.bfloat16), jnp.uint16).astype(jnp.int32)
    hi = lax.bitcast_convert_type(a[:, DH:].astype(jnp.bfloat16), jnp.uint16).astype(jnp.int32)
    return lo | (hi << 16)


def _unpack2(w):
    lo = lax.bitcast_convert_type(w << 16, jnp.float32)
    hi = lax.bitcast_convert_type(w & jnp.int32(-65536), jnp.float32)
    return jnp.concatenate([lo, hi], axis=1).astype(jnp.bfloat16)


def _router_kernel(x_ref, gate_w_ref, sg_ref, su_ref, sd_ref, segw_ref,
                   logits_ref, pos_ref, w_t_ref, be_ref, xp_ref, shared_ref):
    x = x_ref[...]
    logits = jax.lax.dot_general(
        x, gate_w_ref[...], (((1,), (1,)), ((), ())),
        precision=jax.lax.Precision.DEFAULT,
        preferred_element_type=jnp.float32)
    logits_ref[...] = logits

    xp_ref[...] = _to_slabs(_pack2(x))

    x16 = x.astype(jnp.bfloat16)
    sg16 = sg_ref[...].astype(jnp.bfloat16)
    su16 = su_ref[...].astype(jnp.bfloat16)
    sd16 = sd_ref[...].astype(jnp.bfloat16)
    g = jax.lax.dot_general(x16, sg16, (((1,), (1,)), ((), ())),
                            preferred_element_type=jnp.float32)
    u = jax.lax.dot_general(x16, su16, (((1,), (1,)), ((), ())),
                            preferred_element_type=jnp.float32)
    h = (_silu(g) * u).astype(jnp.bfloat16)
    ys = jax.lax.dot_general(h, sd16, (((1,), (1,)), ((), ())),
                             preferred_element_type=jnp.float32)
    sgl = jax.lax.dot_general(x, segw_ref[...], (((1,), (1,)), ((), ())),
                              precision=jax.lax.Precision.DEFAULT,
                              preferred_element_type=jnp.float32)
    shared_ref[...] = jax.nn.sigmoid(sgl) * ys

    lt = logits.T
    m = jnp.max(lt, axis=0, keepdims=True)
    p = jnp.exp(lt - m)
    p = p / jnp.sum(p, axis=0, keepdims=True)
    eio = jax.lax.broadcasted_iota(jnp.int32, p.shape, 0)
    m1 = jnp.max(p, axis=0, keepdims=True)
    i1 = jnp.min(jnp.where(p == m1, eio, E), axis=0, keepdims=True)
    p2 = jnp.where(eio == i1, -jnp.inf, p)
    m2 = jnp.max(p2, axis=0, keepdims=True)
    i2 = jnp.min(jnp.where(p2 == m2, eio, E), axis=0, keepdims=True)
    w_t_ref[...] = jnp.concatenate([m1, m2], axis=0)

    onehot = (jnp.where(eio == i1, 1.0, 0.0) + jnp.where(eio == i2, 1.0, 0.0))
    rank_incl = _cumsum_manual(onehot, axis=1)
    rank = rank_incl - onehot
    counts = rank_incl[:, T - 1:T]
    pad = jnp.floor((counts + (BT - 1.0)) * (1.0 / BT)) * BT
    ends = _cumsum_manual(pad, axis=0)
    offs = ends - pad
    slot = offs + rank
    pos1 = jnp.sum(jnp.where(eio == i1, slot, 0.0), axis=0, keepdims=True).astype(jnp.int32)
    pos2 = jnp.sum(jnp.where(eio == i2, slot, 0.0), axis=0, keepdims=True).astype(jnp.int32)
    rows = [pos1[:, r * 128:(r + 1) * 128] for r in range(T // 128)]
    rows += [pos2[:, r * 128:(r + 1) * 128] for r in range(T // 128)]
    pos_ref[...] = jnp.concatenate(rows, axis=0)

    bio = jax.lax.broadcasted_iota(jnp.int32, (NB, E), 0).astype(jnp.float32) * float(BT)
    ends_row = ends.T
    be = jnp.sum(jnp.where(bio >= ends_row, 1.0, 0.0), axis=1, keepdims=True)
    be_ref[...] = jnp.minimum(be, float(E - 1)).astype(jnp.int32)


def _router(x, gate_w, sgw, suw, sdw, segw):
    return pl.pallas_call(
        _router_kernel,
        grid=(1,),
        in_specs=[
            pl.BlockSpec((T, D), lambda i: (0, 0)),
            pl.BlockSpec((E, D), lambda i: (0, 0)),
            pl.BlockSpec((D_FF, D), lambda i: (0, 0)),
            pl.BlockSpec((D_FF, D), lambda i: (0, 0)),
            pl.BlockSpec((D, D_FF), lambda i: (0, 0)),
            pl.BlockSpec((1, D), lambda i: (0, 0)),
        ],
        out_specs=[
            pl.BlockSpec((T, E), lambda i: (0, 0)),
            pl.BlockSpec((NA // 128, 128), lambda i: (0, 0)),
            pl.BlockSpec((2, T), lambda i: (0, 0)),
            pl.BlockSpec((NB, 1), lambda i: (0, 0)),
            pl.BlockSpec((T, PC, 128), lambda i: (0, 0, 0)),
            pl.BlockSpec((T, D), lambda i: (0, 0)),
        ],
        out_shape=[
            jax.ShapeDtypeStruct((T, E), jnp.float32),
            jax.ShapeDtypeStruct((NA // 128, 128), jnp.int32),
            jax.ShapeDtypeStruct((2, T), jnp.float32),
            jax.ShapeDtypeStruct((NB, 1), jnp.int32),
            jax.ShapeDtypeStruct((T, PC, 128), jnp.int32),
            jax.ShapeDtypeStruct((T, D), jnp.float32),
        ],
    )(x, gate_w, sgw, suw, sdw, segw)


def _sc_dispatch(xp3, pos):
    info = plsc.get_sparse_core_info()
    nw = info.num_cores * info.num_subcores
    bpw = NA // nw
    mesh = plsc.VectorSubcoreMesh(core_axis_name="c", subcore_axis_name="s")

    @functools.partial(
        pl.kernel, mesh=mesh,
        out_type=jax.ShapeDtypeStruct((P, PC, 128), jnp.int32),
        scratch_types=[
            pltpu.VMEM((bpw,), jnp.int32),
            pltpu.VMEM((bpw,), jnp.int32),
            pltpu.VMEM((bpw, PC, 128), jnp.int32),
            pltpu.SemaphoreType.DMA,
            pltpu.SemaphoreType.DMA,
        ],
    )
    def k(x_hbm, pos_hbm, xs_hbm, tok_v, pos_v, rows_v, sem_g, sem_s):
        wid = lax.axis_index("s") * info.num_cores + lax.axis_index("c")
        base = wid * bpw
        tbase = jnp.where(base >= T, base - T, base)
        for c in range(bpw // 16):
            tok_v[pl.ds(c * 16, 16)] = tbase + c * 16 + lax.iota(jnp.int32, 16)
        pltpu.sync_copy(pos_hbm.at[pl.ds(base, bpw)], pos_v)
        pltpu.async_copy(x_hbm.at[tok_v], rows_v, sem_g).wait()
        pltpu.async_copy(rows_v, xs_hbm.at[pos_v], sem_s).wait()

    return k(xp3, pos)


def _ffn_kernel(be_ref, xs_ref, wg_ref, wu_ref, wd_ref, y_ref):
    xb = _unpack2(_from_slabs(xs_ref[...]))
    wg16 = wg_ref[0]
    wu16 = wu_ref[0]
    wd16 = wd_ref[0]
    g = jax.lax.dot_general(xb, wg16, (((1,), (1,)), ((), ())),
                            preferred_element_type=jnp.float32)
    u = jax.lax.dot_general(xb, wu16, (((1,), (1,)), ((), ())),
                            preferred_element_type=jnp.float32)
    h = (_silu(g) * u).astype(jnp.bfloat16)
    y = jax.lax.dot_general(h, wd16, (((1,), (1,)), ((), ())),
                            preferred_element_type=jnp.float32)
    y_ref[...] = _to_slabs(_pack2(y))


def _expert_ffn(xs3, be, wg, wu, wd):
    return pl.pallas_call(
        _ffn_kernel,
        grid_spec=pltpu.PrefetchScalarGridSpec(
            num_scalar_prefetch=1,
            grid=(NB,),
            in_specs=[
                pl.BlockSpec((BT, PC, 128), lambda b, be_ref: (b, 0, 0)),
                pl.BlockSpec((1, D_FF, D), lambda b, be_ref: (be_ref[b], 0, 0)),
                pl.BlockSpec((1, D_FF, D), lambda b, be_ref: (be_ref[b], 0, 0)),
                pl.BlockSpec((1, D, D_FF), lambda b, be_ref: (be_ref[b], 0, 0)),
            ],
            out_specs=pl.BlockSpec((BT, PC, 128), lambda b, be_ref: (b, 0, 0)),
        ),
        out_shape=jax.ShapeDtypeStruct((P, PC, 128), jnp.int32),
    )(be, xs3, wg, wu, wd)


def _sc_combine(y3, pos):
    info = plsc.get_sparse_core_info()
    nw = info.num_cores * info.num_subcores
    bpw = NA // nw
    mesh = plsc.VectorSubcoreMesh(core_axis_name="c", subcore_axis_name="s")

    @functools.partial(
        pl.kernel, mesh=mesh,
        out_type=jax.ShapeDtypeStruct((NA, PC, 128), jnp.int32),
        scratch_types=[
            pltpu.VMEM((bpw,), jnp.int32),
            pltpu.VMEM((bpw, PC, 128), jnp.int32),
            pltpu.SemaphoreType.DMA,
        ],
    )
    def k(y_hbm, pos_hbm, out_hbm, pos_v, rows_v, sem):
        wid = lax.axis_index("s") * info.num_cores + lax.axis_index("c")
        base = wid * bpw
        pltpu.sync_copy(pos_hbm.at[pl.ds(base, bpw)], pos_v)
        pltpu.async_copy(y_hbm.at[pos_v], rows_v, sem).wait()
        pltpu.sync_copy(rows_v, out_hbm.at[pl.ds(base, bpw)])

    return k(y3, pos)


TB5 = 512


def _final_kernel(y0_ref, y1_ref, w_t_ref, shared_ref, out_ref):
    y0 = _unpack2(_from_slabs(y0_ref[...])).astype(jnp.float32)
    y1 = _unpack2(_from_slabs(y1_ref[...])).astype(jnp.float32)
    w = w_t_ref[...]
    w0 = w[0:1, :].T
    w1 = w[1:2, :].T
    out_ref[...] = w0 * y0 + w1 * y1 + shared_ref[...]


def _final(y01, w_t, shared):
    n_tb = T // TB5
    return pl.pallas_call(
        _final_kernel,
        grid=(n_tb,),
        in_specs=[
            pl.BlockSpec((TB5, PC, 128), lambda t: (t, 0, 0)),
            pl.BlockSpec((TB5, PC, 128), lambda t: (t + T // TB5, 0, 0)),
            pl.BlockSpec((2, TB5), lambda t: (0, t)),
            pl.BlockSpec((TB5, D), lambda t: (t, 0)),
        ],
        out_specs=pl.BlockSpec((TB5, D), lambda t: (t, 0)),
        out_shape=jax.ShapeDtypeStruct((T, D), jnp.float32),
    )(y01, y01, w_t, shared)


def kernel(hidden_states, gate_w, expert_gate_w, expert_up_w, expert_down_w,
           shared_gate_w, shared_up_w, shared_down_w, shared_expert_gate_w):
    b, s, d = hidden_states.shape
    x = hidden_states.reshape(-1, d)

    logits, pos_p, w_t, be, xp, shared = _router(
        x, gate_w, shared_gate_w, shared_up_w, shared_down_w, shared_expert_gate_w)
    pos = pos_p.reshape(NA)
    xs = _sc_dispatch(xp, pos)
    y = _expert_ffn(xs, be.reshape(NB),
                    expert_gate_w.astype(jnp.bfloat16),
                    expert_up_w.astype(jnp.bfloat16),
                    expert_down_w.astype(jnp.bfloat16))
    y01 = _sc_combine(y, pos)
    out = _final(y01, w_t, shared)
    return (out.reshape(b, s, d), logits)


# speedup vs baseline: 1.1259x; 1.1259x over previous
import functools

import jax
import jax.numpy as jnp
from jax import lax
from jax.experimental import pallas as pl
from jax.experimental.pallas import tpu as pltpu
from jax.experimental.pallas import tpu_sc as plsc

E = 8
TOP_K = 2
D = 1024
DH = D // 2
PC = DH // 128
D_FF = 512
T = 2048
BT = 256
P = 6144
NB = P // BT
NA = T * TOP_K


def _silu(x):
    return x * jax.nn.sigmoid(x)


def _cumsum_manual(y, axis):
    n = y.shape[axis]
    s = 1
    while s < n:
        if axis == 1:
            sh = jnp.concatenate([jnp.zeros((y.shape[0], s), y.dtype), y[:, :-s]], axis=1)
        else:
            sh = jnp.concatenate([jnp.zeros((s, y.shape[1]), y.dtype), y[:-s, :]], axis=0)
        y = y + sh
        s *= 2
    return y


def _to_slabs(a):
    w = a.shape[1]
    return jnp.concatenate(
        [a[:, None, c * 128:(c + 1) * 128] for c in range(w // 128)], axis=1)


def _from_slabs(a3):
    return jnp.concatenate([a3[:, c, :] for c in range(a3.shape[1])], axis=1)


def _pack2(a):
    lo = lax.bitcast_convert_type(a[:, :DH].astype(jnp.bfloat16), jnp.uint16).astype(jnp.int32)
    hi = lax.bitcast_convert_type(a[:, DH:].astype(jnp.bfloat16), jnp.uint16).astype(jnp.int32)
    return lo | (hi << 16)


def _unpack2(w):
    lo = lax.bitcast_convert_type(w << 16, jnp.float32)
    hi = lax.bitcast_convert_type(w & jnp.int32(-65536), jnp.float32)
    return jnp.concatenate([lo, hi], axis=1).astype(jnp.bfloat16)


def _router_kernel(x_ref, gate_w_ref,
                   logits_ref, pos_ref, w_t_ref, be_ref, xp_ref):
    x = x_ref[...]
    logits = jax.lax.dot_general(
        x, gate_w_ref[...], (((1,), (1,)), ((), ())),
        precision=jax.lax.Precision.DEFAULT,
        preferred_element_type=jnp.float32)
    logits_ref[...] = logits

    xp_ref[...] = _to_slabs(_pack2(x))

    lt = logits.T
    m = jnp.max(lt, axis=0, keepdims=True)
    p = jnp.exp(lt - m)
    p = p / jnp.sum(p, axis=0, keepdims=True)
    eio = jax.lax.broadcasted_iota(jnp.int32, p.shape, 0)
    m1 = jnp.max(p, axis=0, keepdims=True)
    i1 = jnp.min(jnp.where(p == m1, eio, E), axis=0, keepdims=True)
    p2 = jnp.where(eio == i1, -jnp.inf, p)
    m2 = jnp.max(p2, axis=0, keepdims=True)
    i2 = jnp.min(jnp.where(p2 == m2, eio, E), axis=0, keepdims=True)
    w_t_ref[...] = jnp.concatenate([m1, m2], axis=0)

    onehot = (jnp.where(eio == i1, 1.0, 0.0) + jnp.where(eio == i2, 1.0, 0.0))
    rank_incl = _cumsum_manual(onehot, axis=1)
    rank = rank_incl - onehot
    counts = rank_incl[:, T - 1:T]
    pad = jnp.floor((counts + (BT - 1.0)) * (1.0 / BT)) * BT
    ends = _cumsum_manual(pad, axis=0)
    offs = ends - pad
    slot = offs + rank
    pos1 = jnp.sum(jnp.where(eio == i1, slot, 0.0), axis=0, keepdims=True).astype(jnp.int32)
    pos2 = jnp.sum(jnp.where(eio == i2, slot, 0.0), axis=0, keepdims=True).astype(jnp.int32)
    rows = [pos1[:, r * 128:(r + 1) * 128] for r in range(T // 128)]
    rows += [pos2[:, r * 128:(r + 1) * 128] for r in range(T // 128)]
    pos_ref[...] = jnp.concatenate(rows, axis=0)

    bio = jax.lax.broadcasted_iota(jnp.int32, (NB, E), 0).astype(jnp.float32) * float(BT)
    ends_row = ends.T
    be = jnp.sum(jnp.where(bio >= ends_row, 1.0, 0.0), axis=1, keepdims=True)
    be_ref[...] = jnp.minimum(be, float(E - 1)).astype(jnp.int32)


def _router(x, gate_w):
    return pl.pallas_call(
        _router_kernel,
        grid=(1,),
        in_specs=[
            pl.BlockSpec((T, D), lambda i: (0, 0)),
            pl.BlockSpec((E, D), lambda i: (0, 0)),
        ],
        out_specs=[
            pl.BlockSpec((T, E), lambda i: (0, 0)),
            pl.BlockSpec((NA // 128, 128), lambda i: (0, 0)),
            pl.BlockSpec((2, T), lambda i: (0, 0)),
            pl.BlockSpec((NB, 1), lambda i: (0, 0)),
            pl.BlockSpec((T, PC, 128), lambda i: (0, 0, 0)),
        ],
        out_shape=[
            jax.ShapeDtypeStruct((T, E), jnp.float32),
            jax.ShapeDtypeStruct((NA // 128, 128), jnp.int32),
            jax.ShapeDtypeStruct((2, T), jnp.float32),
            jax.ShapeDtypeStruct((NB, 1), jnp.int32),
            jax.ShapeDtypeStruct((T, PC, 128), jnp.int32),
        ],
    )(x, gate_w)


TBS = 1024


def _shared_kernel(x_ref, sg_ref, su_ref, sd_ref, segw_ref, shared_ref):
    x = x_ref[...]
    x16 = x.astype(jnp.bfloat16)
    sg16 = sg_ref[...].astype(jnp.bfloat16)
    su16 = su_ref[...].astype(jnp.bfloat16)
    sd16 = sd_ref[...].astype(jnp.bfloat16)
    g = jax.lax.dot_general(x16, sg16, (((1,), (1,)), ((), ())),
                            preferred_element_type=jnp.float32)
    u = jax.lax.dot_general(x16, su16, (((1,), (1,)), ((), ())),
                            preferred_element_type=jnp.float32)
    h = (_silu(g) * u).astype(jnp.bfloat16)
    ys = jax.lax.dot_general(h, sd16, (((1,), (1,)), ((), ())),
                             preferred_element_type=jnp.float32)
    sgl = jax.lax.dot_general(x, segw_ref[...], (((1,), (1,)), ((), ())),
                              precision=jax.lax.Precision.DEFAULT,
                              preferred_element_type=jnp.float32)
    shared_ref[...] = jax.nn.sigmoid(sgl) * ys


def _shared(x, sgw, suw, sdw, segw):
    return pl.pallas_call(
        _shared_kernel,
        grid=(T // TBS,),
        in_specs=[
            pl.BlockSpec((TBS, D), lambda t: (t, 0)),
            pl.BlockSpec((D_FF, D), lambda t: (0, 0)),
            pl.BlockSpec((D_FF, D), lambda t: (0, 0)),
            pl.BlockSpec((D, D_FF), lambda t: (0, 0)),
            pl.BlockSpec((1, D), lambda t: (0, 0)),
        ],
        out_specs=pl.BlockSpec((TBS, D), lambda t: (t, 0)),
        out_shape=jax.ShapeDtypeStruct((T, D), jnp.float32),
    )(x, sgw, suw, sdw, segw)


def _sc_dispatch(xp3, pos):
    info = plsc.get_sparse_core_info()
    nw = info.num_cores * info.num_subcores
    bpw = NA // nw
    mesh = plsc.VectorSubcoreMesh(core_axis_name="c", subcore_axis_name="s")

    @functools.partial(
        pl.kernel, mesh=mesh,
        out_type=jax.ShapeDtypeStruct((P, PC, 128), jnp.int32),
        scratch_types=[
            pltpu.VMEM((bpw,), jnp.int32),
            pltpu.VMEM((bpw,), jnp.int32),
            pltpu.VMEM((bpw, PC, 128), jnp.int32),
            pltpu.SemaphoreType.DMA,
            pltpu.SemaphoreType.DMA,
        ],
    )
    def k(x_hbm, pos_hbm, xs_hbm, tok_v, pos_v, rows_v, sem_g, sem_s):
        wid = lax.axis_index("s") * info.num_cores + lax.axis_index("c")
        base = wid * bpw
        tbase = jnp.where(base >= T, base - T, base)
        for c in range(bpw // 16):
            tok_v[pl.ds(c * 16, 16)] = tbase + c * 16 + lax.iota(jnp.int32, 16)
        pltpu.sync_copy(pos_hbm.at[pl.ds(base, bpw)], pos_v)
        pltpu.async_copy(x_hbm.at[tok_v], rows_v, sem_g).wait()
        pltpu.async_copy(rows_v, xs_hbm.at[pos_v], sem_s).wait()

    return k(xp3, pos)


def _ffn_kernel(be_ref, xs_ref, wg_ref, wu_ref, wd_ref, y_ref):
    xb = _unpack2(_from_slabs(xs_ref[...]))
    wg16 = wg_ref[0].astype(jnp.bfloat16)
    wu16 = wu_ref[0].astype(jnp.bfloat16)
    wd16 = wd_ref[0].astype(jnp.bfloat16)
    g = jax.lax.dot_general(xb, wg16, (((1,), (1,)), ((), ())),
                            preferred_element_type=jnp.float32)
    u = jax.lax.dot_general(xb, wu16, (((1,), (1,)), ((), ())),
                            preferred_element_type=jnp.float32)
    h = (_silu(g) * u).astype(jnp.bfloat16)
    y = jax.lax.dot_general(h, wd16, (((1,), (1,)), ((), ())),
                            preferred_element_type=jnp.float32)
    y_ref[...] = _to_slabs(_pack2(y))


def _expert_ffn(xs3, be, wg, wu, wd):
    return pl.pallas_call(
        _ffn_kernel,
        grid_spec=pltpu.PrefetchScalarGridSpec(
            num_scalar_prefetch=1,
            grid=(NB,),
            in_specs=[
                pl.BlockSpec((BT, PC, 128), lambda b, be_ref: (b, 0, 0)),
                pl.BlockSpec((1, D_FF, D), lambda b, be_ref: (be_ref[b], 0, 0)),
                pl.BlockSpec((1, D_FF, D), lambda b, be_ref: (be_ref[b], 0, 0)),
                pl.BlockSpec((1, D, D_FF), lambda b, be_ref: (be_ref[b], 0, 0)),
            ],
            out_specs=pl.BlockSpec((BT, PC, 128), lambda b, be_ref: (b, 0, 0)),
        ),
        out_shape=jax.ShapeDtypeStruct((P, PC, 128), jnp.int32),
    )(be, xs3, wg, wu, wd)


def _sc_combine(y3, pos):
    info = plsc.get_sparse_core_info()
    nw = info.num_cores * info.num_subcores
    bpw = NA // nw
    mesh = plsc.VectorSubcoreMesh(core_axis_name="c", subcore_axis_name="s")

    @functools.partial(
        pl.kernel, mesh=mesh,
        out_type=jax.ShapeDtypeStruct((NA, PC, 128), jnp.int32),
        scratch_types=[
            pltpu.VMEM((bpw,), jnp.int32),
            pltpu.VMEM((bpw, PC, 128), jnp.int32),
            pltpu.SemaphoreType.DMA,
        ],
    )
    def k(y_hbm, pos_hbm, out_hbm, pos_v, rows_v, sem):
        wid = lax.axis_index("s") * info.num_cores + lax.axis_index("c")
        base = wid * bpw
        pltpu.sync_copy(pos_hbm.at[pl.ds(base, bpw)], pos_v)
        pltpu.async_copy(y_hbm.at[pos_v], rows_v, sem).wait()
        pltpu.sync_copy(rows_v, out_hbm.at[pl.ds(base, bpw)])

    return k(y3, pos)


TB5 = 512


def _final_kernel(y0_ref, y1_ref, w_t_ref, shared_ref, out_ref):
    y0 = _unpack2(_from_slabs(y0_ref[...])).astype(jnp.float32)
    y1 = _unpack2(_from_slabs(y1_ref[...])).astype(jnp.float32)
    w = w_t_ref[...]
    w0 = w[0:1, :].T
    w1 = w[1:2, :].T
    out_ref[...] = w0 * y0 + w1 * y1 + shared_ref[...]


def _final(y01, w_t, shared):
    n_tb = T // TB5
    return pl.pallas_call(
        _final_kernel,
        grid=(n_tb,),
        in_specs=[
            pl.BlockSpec((TB5, PC, 128), lambda t: (t, 0, 0)),
            pl.BlockSpec((TB5, PC, 128), lambda t: (t + T // TB5, 0, 0)),
            pl.BlockSpec((2, TB5), lambda t: (0, t)),
            pl.BlockSpec((TB5, D), lambda t: (t, 0)),
        ],
        out_specs=pl.BlockSpec((TB5, D), lambda t: (t, 0)),
        out_shape=jax.ShapeDtypeStruct((T, D), jnp.float32),
    )(y01, y01, w_t, shared)


def kernel(hidden_states, gate_w, expert_gate_w, expert_up_w, expert_down_w,
           shared_gate_w, shared_up_w, shared_down_w, shared_expert_gate_w):
    b, s, d = hidden_states.shape
    x = hidden_states.reshape(-1, d)

    logits, pos_p, w_t, be, xp = _router(x, gate_w)
    pos = pos_p.reshape(NA)
    xs = _sc_dispatch(xp, pos)
    shared = _shared(x, shared_gate_w, shared_up_w, shared_down_w,
                     shared_expert_gate_w)
    y = _expert_ffn(xs, be.reshape(NB), expert_gate_w, expert_up_w, expert_down_w)
    y01 = _sc_combine(y, pos)
    out = _final(y01, w_t, shared)
    return (out.reshape(b, s, d), logits)
